# baseline (speedup 1.0000x reference)
import jax
import jax.numpy as jnp
from jax.experimental import pallas as pl
from jax.experimental.pallas import tpu as pltpu

_N = 1000
_B = 256
_RB = 8
_GRID = _N // _RB
_NBC = _B // _RB


def _enqueue_kernel(comp_hbm, mem_hbm, out_ref, sem):
    i = pl.program_id(0)

    @pl.when(i < _NBC)
    def _():
        pltpu.make_async_copy(
            comp_hbm.at[pl.ds(i * _RB, _RB)], out_ref, sem).start()
        pltpu.make_async_copy(
            comp_hbm.at[pl.ds(i * _RB, _RB)], out_ref, sem).wait()

    @pl.when(i >= _NBC)
    def _():
        pltpu.make_async_copy(
            mem_hbm.at[pl.ds(i * _RB, _RB)], out_ref, sem).start()
        pltpu.make_async_copy(
            mem_hbm.at[pl.ds(i * _RB, _RB)], out_ref, sem).wait()


def kernel(memory_bank, components):
    comps = jax.lax.stop_gradient(components)
    return pl.pallas_call(
        _enqueue_kernel,
        grid=(_GRID,),
        in_specs=[
            pl.BlockSpec(memory_space=pltpu.MemorySpace.HBM),
            pl.BlockSpec(memory_space=pltpu.MemorySpace.HBM),
        ],
        out_specs=pl.BlockSpec((_RB, 256, 256), lambda i: (i, 0, 0)),
        out_shape=jax.ShapeDtypeStruct((_N, 256, 256), memory_bank.dtype),
        scratch_shapes=[pltpu.SemaphoreType.DMA],
    )(comps, memory_bank)


# speedup vs baseline: 1.7026x; 1.7026x over previous
import jax
import jax.numpy as jnp
from jax.experimental import pallas as pl
from jax.experimental.pallas import tpu as pltpu

_N = 1000
_B = 256
_R = 8
_NB = _N // _R
_NB_COMP = _B // _R


def _enqueue_kernel(comp_ref, mem_ref, out_ref):
    i = pl.program_id(0)

    @pl.when(i < _NB_COMP)
    def _():
        out_ref[...] = comp_ref[...]

    @pl.when(i >= _NB_COMP)
    def _():
        out_ref[...] = mem_ref[...]


def kernel(memory_bank, components):
    comps = jax.lax.stop_gradient(components)
    return pl.pallas_call(
        _enqueue_kernel,
        grid=(_NB,),
        in_specs=[
            pl.BlockSpec((_R, 256, 256), lambda i: (jnp.minimum(i, _NB_COMP - 1), 0, 0)),
            pl.BlockSpec((_R, 256, 256), lambda i: (jnp.maximum(i, _NB_COMP), 0, 0)),
        ],
        out_specs=pl.BlockSpec((_R, 256, 256), lambda i: (i, 0, 0)),
        out_shape=jax.ShapeDtypeStruct((_N, 256, 256), memory_bank.dtype),
        compiler_params=pltpu.CompilerParams(
            dimension_semantics=("parallel",),
        ),
    )(comps, memory_bank)


# speedup vs baseline: 1.8979x; 1.1147x over previous
import jax
import jax.numpy as jnp
from jax.experimental import pallas as pl
from jax.experimental.pallas import tpu as pltpu

_N = 1000
_B = 256
_RC = 16
_NBUF = 8
_RAHEAD = 4

_CHUNKS = (
    [("c", r, _RC) for r in range(0, _B, _RC)]
    + [("m", r, _RC) for r in range(_B, _N - 8, _RC)]
    + [("m", _N - 8, 8)]
)
_NCH = len(_CHUNKS)


def _enqueue_kernel(comp_hbm, mem_hbm, out_hbm, buf, rsem, wsem):
    def rd(i, s):
        src, r0, nr = _CHUNKS[i]
        ref = comp_hbm if src == "c" else mem_hbm
        return pltpu.make_async_copy(
            ref.at[pl.ds(r0, nr)], buf.at[s, pl.ds(0, nr)], rsem.at[s])

    def wr(i, s):
        _, r0, nr = _CHUNKS[i]
        return pltpu.make_async_copy(
            buf.at[s, pl.ds(0, nr)], out_hbm.at[pl.ds(r0, nr)], wsem.at[s])

    for i in range(_RAHEAD):
        rd(i, i % _NBUF).start()
    for i in range(_NCH):
        s = i % _NBUF
        rd(i, s).wait()
        wr(i, s).start()
        ni = i + _RAHEAD
        if ni < _NCH:
            ns = ni % _NBUF
            if ni >= _NBUF:
                wr(ni - _NBUF, ns).wait()
            rd(ni, ns).start()
    for i in range(_NCH - _NBUF, _NCH):
        wr(i, i % _NBUF).wait()


def kernel(memory_bank, components):
    comps = jax.lax.stop_gradient(components)
    return pl.pallas_call(
        _enqueue_kernel,
        in_specs=[
            pl.BlockSpec(memory_space=pltpu.MemorySpace.HBM),
            pl.BlockSpec(memory_space=pltpu.MemorySpace.HBM),
        ],
        out_specs=pl.BlockSpec(memory_space=pltpu.MemorySpace.HBM),
        out_shape=jax.ShapeDtypeStruct((_N, 256, 256), memory_bank.dtype),
        scratch_shapes=[
            pltpu.VMEM((_NBUF, _RC, 256, 256), jnp.float32),
            pltpu.SemaphoreType.DMA((_NBUF,)),
            pltpu.SemaphoreType.DMA((_NBUF,)),
        ],
    )(comps, memory_bank)
